# Initial kernel scaffold; baseline (speedup 1.0000x reference)
import jax
import jax.numpy as jnp
from jax.experimental import pallas as pl


def kernel(x, edge_index, edge_attr, batch, W_self, W_msg, W_edge, b_msg, Wp, bp):
    raise NotImplementedError("write your pallas kernel here")


# speedup vs baseline: 6.3159x; 6.3159x over previous
import functools

import jax
import jax.numpy as jnp
from jax import lax
from jax.experimental import pallas as pl
from jax.experimental.pallas import tpu as pltpu
from jax.experimental.pallas import tpu_sc as plsc

_N = 10000
_E = 320000
_D = 128
_DE = 16
_H = 128
_G = 64

_NC = 2
_NS = 16
_EPT = _E // (_NC * _NS)
_CH = 80
_NCH = _EPT // _CH
_NP = 10240
_RPT = _NP // _NS
_NZ = _RPT // _CH

_R = 1000
_NB = _N // _R

_F32 = jnp.float32
_HI = lax.Precision.HIGHEST


def _sc_body(x_hbm, src_hbm, dst_hbm, ea_hbm, aggx_out, agge_out,
             sidx_a, sidx_b, didx_a, didx_b, rows_a, rows_b, ea_a, ea_b,
             aggx_sh, agge_sh,
             ss_a, ss_b, sd_a, sd_b, se_a, se_b, sg_a, sg_b):
    cc = lax.axis_index("c")
    ss = lax.axis_index("s")
    eb = (cc * _NS + ss) * _EPT
    rb = ss * _RPT

    def _zr(i, _):
        rows_a[i // 8, pl.ds((i % 8) * 16, 16)] = jnp.zeros((16,), _F32)
        return 0

    lax.fori_loop(0, _CH * 8, _zr, 0)

    def _ze(i, _):
        ea_a[i, :] = jnp.zeros((16,), _F32)
        return 0

    lax.fori_loop(0, _CH, _ze, 0)

    def _zs(k, _):
        pltpu.sync_copy(rows_a, aggx_sh.at[pl.ds(rb + k * _CH, _CH)])
        pltpu.sync_copy(ea_a, agge_sh.at[pl.ds(rb + k * _CH, _CH)])
        return 0

    lax.fori_loop(0, _NZ, _zs, 0)
    plsc.subcore_barrier()

    def _start_inputs(j, sidx, didx, ea, s_s, s_d, s_e):
        pltpu.async_copy(src_hbm.at[pl.ds(eb + j * _CH, _CH)], sidx, s_s)
        pltpu.async_copy(dst_hbm.at[pl.ds(eb + j * _CH, _CH)], didx, s_d)
        pltpu.async_copy(ea_hbm.at[pl.ds(eb + j * _CH, _CH)], ea, s_e)

    def _wait_idx(sidx, s_s):
        pltpu.make_async_copy(src_hbm.at[pl.ds(0, _CH)], sidx, s_s).wait()

    def _start_gather(sidx, rows, s_g):
        pltpu.async_copy(x_hbm.at[sidx], rows, s_g)

    def _drain(didx, rows, ea, s_d, s_e, s_g):
        pltpu.make_async_copy(x_hbm.at[pl.ds(0, _CH)], rows, s_g).wait()
        pltpu.make_async_copy(dst_hbm.at[pl.ds(0, _CH)], didx, s_d).wait()
        pltpu.make_async_copy(ea_hbm.at[pl.ds(0, _CH)], ea, s_e).wait()
        pltpu.sync_copy(rows, aggx_sh.at[didx], add=True)
        pltpu.sync_copy(ea, agge_sh.at[didx], add=True)

    _start_inputs(0, sidx_a, didx_a, ea_a, ss_a, sd_a, se_a)
    _wait_idx(sidx_a, ss_a)
    _start_gather(sidx_a, rows_a, sg_a)
    _start_inputs(1, sidx_b, didx_b, ea_b, ss_b, sd_b, se_b)

    def _pair(jj, _):
        j0 = 2 * jj
        _wait_idx(sidx_b, ss_b)
        _start_gather(sidx_b, rows_b, sg_b)
        _drain(didx_a, rows_a, ea_a, sd_a, se_a, sg_a)
        _start_inputs(j0 + 2, sidx_a, didx_a, ea_a, ss_a, sd_a, se_a)
        _wait_idx(sidx_a, ss_a)
        _start_gather(sidx_a, rows_a, sg_a)
        _drain(didx_b, rows_b, ea_b, sd_b, se_b, sg_b)

        @pl.when(j0 + 3 < _NCH)
        def _():
            _start_inputs(j0 + 3, sidx_b, didx_b, ea_b, ss_b, sd_b, se_b)

        return 0

    lax.fori_loop(0, (_NCH - 1) // 2, _pair, 0)
    _drain(didx_a, rows_a, ea_a, sd_a, se_a, sg_a)
    plsc.subcore_barrier()

    def _wb(k, _):
        r = rb + k * _CH
        pltpu.sync_copy(aggx_sh.at[pl.ds(r, _CH)], rows_a)
        pltpu.sync_copy(rows_a, aggx_out.at[cc, pl.ds(r, _CH)])
        pltpu.sync_copy(agge_sh.at[pl.ds(r, _CH)], ea_a)
        pltpu.sync_copy(ea_a, agge_out.at[cc, pl.ds(r, _CH)])
        return 0

    lax.fori_loop(0, _NZ, _wb, 0)


_sc_agg = functools.partial(
    pl.kernel,
    out_type=[
        jax.ShapeDtypeStruct((_NC, _NP, _D), _F32),
        jax.ShapeDtypeStruct((_NC, _NP, _DE), _F32),
    ],
    mesh=plsc.VectorSubcoreMesh(core_axis_name="c", subcore_axis_name="s"),
    compiler_params=pltpu.CompilerParams(use_tc_tiling_on_sc=False),
    scratch_types=[
        pltpu.VMEM((_CH,), jnp.int32),
        pltpu.VMEM((_CH,), jnp.int32),
        pltpu.VMEM((_CH,), jnp.int32),
        pltpu.VMEM((_CH,), jnp.int32),
        pltpu.VMEM((_CH, _D), _F32),
        pltpu.VMEM((_CH, _D), _F32),
        pltpu.VMEM((_CH, _DE), _F32),
        pltpu.VMEM((_CH, _DE), _F32),
        pltpu.VMEM_SHARED((_NP, _D), _F32),
        pltpu.VMEM_SHARED((_NP, _DE), _F32),
        pltpu.SemaphoreType.DMA,
        pltpu.SemaphoreType.DMA,
        pltpu.SemaphoreType.DMA,
        pltpu.SemaphoreType.DMA,
        pltpu.SemaphoreType.DMA,
        pltpu.SemaphoreType.DMA,
        pltpu.SemaphoreType.DMA,
        pltpu.SemaphoreType.DMA,
    ],
)(_sc_body)


def _tc_body(xr, a0r, a1r, e0r, e1r, br, wsr, wmr, wer, bmr, wpr, bpr,
             outr, gsumr, cntr):
    i = pl.program_id(0)

    @pl.when(i == 0)
    def _init():
        gsumr[...] = jnp.zeros_like(gsumr)
        cntr[...] = jnp.zeros_like(cntr)

    h = (jnp.dot(xr[...], wsr[...], precision=_HI, preferred_element_type=_F32)
         + jnp.dot(a0r[0] + a1r[0], wmr[...], precision=_HI,
                   preferred_element_type=_F32)
         + jnp.dot(e0r[0] + e1r[0], wer[...], precision=_HI,
                   preferred_element_type=_F32)
         + bmr[...])
    h = jnp.where(h > 0, h, 0.01 * h)

    oht = (br[0] == lax.broadcasted_iota(jnp.int32, (_G, _R), 0)).astype(_F32)
    gsumr[...] += jnp.dot(oht, h, precision=_HI, preferred_element_type=_F32)
    cntr[...] += jnp.dot(oht, jnp.ones((_R, _D), _F32), precision=_HI,
                         preferred_element_type=_F32)

    @pl.when(i == _NB - 1)
    def _fin():
        gmean = gsumr[...] / jnp.maximum(cntr[...], 1.0)
        n2 = jnp.sum(gmean * gmean, axis=1, keepdims=True)
        nrm = jnp.maximum(jnp.sqrt(n2), 1e-12)
        embs = (gmean / nrm).astype(jnp.bfloat16).astype(_F32)
        wp16 = wpr[...].astype(jnp.bfloat16).astype(_F32)
        outr[...] = jnp.sum(embs * wp16, axis=1, keepdims=True) + bpr[...]


_tc_head = pl.pallas_call(
    _tc_body,
    grid=(_NB,),
    in_specs=[
        pl.BlockSpec((_R, _D), lambda i: (i, 0)),
        pl.BlockSpec((1, _R, _D), lambda i: (0, i, 0)),
        pl.BlockSpec((1, _R, _D), lambda i: (1, i, 0)),
        pl.BlockSpec((1, _R, _DE), lambda i: (0, i, 0)),
        pl.BlockSpec((1, _R, _DE), lambda i: (1, i, 0)),
        pl.BlockSpec((1, 1, _R), lambda i: (i, 0, 0)),
        pl.BlockSpec((_D, _H), lambda i: (0, 0)),
        pl.BlockSpec((_D, _H), lambda i: (0, 0)),
        pl.BlockSpec((_DE, _H), lambda i: (0, 0)),
        pl.BlockSpec((1, _H), lambda i: (0, 0)),
        pl.BlockSpec((1, _H), lambda i: (0, 0)),
        pl.BlockSpec((1, 1), lambda i: (0, 0)),
    ],
    out_specs=pl.BlockSpec((_G, 1), lambda i: (0, 0)),
    out_shape=jax.ShapeDtypeStruct((_G, 1), _F32),
    scratch_shapes=[
        pltpu.VMEM((_G, _D), _F32),
        pltpu.VMEM((_G, _D), _F32),
    ],
)


def kernel(x, edge_index, edge_attr, batch, W_self, W_msg, W_edge, b_msg,
           Wp, bp):
    aggx, agge = _sc_agg(x, edge_index[0], edge_index[1], edge_attr)
    return _tc_head(x, aggx, aggx, agge, agge, batch.reshape(_NB, 1, _R),
                    W_self, W_msg, W_edge, b_msg.reshape(1, _H),
                    Wp.reshape(1, _H), bp.reshape(1, 1))
